# Initial kernel scaffold; baseline (speedup 1.0000x reference)
import jax
import jax.numpy as jnp
from jax.experimental import pallas as pl


def kernel(walks, table):
    raise NotImplementedError("write your pallas kernel here")


# speedup vs baseline: 1.1705x; 1.1705x over previous
import functools

import jax
import jax.numpy as jnp
from jax import lax
from jax.experimental import pallas as pl
from jax.experimental.pallas import tpu as pltpu
from jax.experimental.pallas import tpu_sc as plsc

_NC = 2
_NS = 16
_NW = _NC * _NS
_LANE = 128
_ROWS_PER_STEP = 8


@functools.lru_cache(maxsize=None)
def _make_gather(B, V, D, dtype):
    b_per_w = B // _NW
    chunk = _LANE * _ROWS_PER_STEP
    n_steps = b_per_w // chunk
    mesh = plsc.VectorSubcoreMesh(core_axis_name="c", subcore_axis_name="s")

    @functools.partial(
        pl.kernel,
        out_type=jax.ShapeDtypeStruct((B, D), dtype),
        mesh=mesh,
        scratch_types=[
            pltpu.VMEM((_ROWS_PER_STEP, _LANE), jnp.int32),
            pltpu.VMEM((chunk, D), dtype),
            pltpu.SemaphoreType.DMA,
        ],
        compiler_params=pltpu.CompilerParams(use_tc_tiling_on_sc=False),
    )
    def gather_kernel(idx_hbm, table_hbm, out_hbm, idx_v, rows_v, sem):
        wid = lax.axis_index("s") * _NC + lax.axis_index("c")
        row_base = wid * (b_per_w // _LANE)
        out_base = wid * b_per_w

        def step(i, carry):
            r0 = row_base + i * _ROWS_PER_STEP
            pltpu.sync_copy(idx_hbm.at[pl.ds(r0, _ROWS_PER_STEP)], idx_v)
            waits = []
            for j in range(_ROWS_PER_STEP):
                waits.append(
                    pltpu.async_copy(
                        table_hbm.at[idx_v.at[j]],
                        rows_v.at[pl.ds(j * _LANE, _LANE)],
                        sem,
                    )
                )
            for w in waits:
                w.wait()
            pltpu.sync_copy(
                rows_v, out_hbm.at[pl.ds(out_base + i * chunk, chunk)]
            )
            return carry

        lax.fori_loop(0, n_steps, step, 0)

    return gather_kernel


def kernel(walks, table):
    (B,) = walks.shape
    V, D = table.shape
    idx2d = walks.astype(jnp.int32).reshape(B // _LANE, _LANE)
    return _make_gather(B, V, D, table.dtype)(idx2d, table)


# speedup vs baseline: 1.2029x; 1.0277x over previous
import functools

import jax
import jax.numpy as jnp
from jax import lax
from jax.experimental import pallas as pl
from jax.experimental.pallas import tpu as pltpu
from jax.experimental.pallas import tpu_sc as plsc

_NC = 2
_NS = 16
_NW = _NC * _NS
_LANE = 128
_ROWS_PER_STEP = 8
_NBUF = 2


@functools.lru_cache(maxsize=None)
def _make_gather(B, V, D, dtype):
    b_per_w = B // _NW
    rows_per_w = b_per_w // _LANE
    chunk = _LANE * _ROWS_PER_STEP
    n_steps = b_per_w // chunk
    n_outer = n_steps // _NBUF
    mesh = plsc.VectorSubcoreMesh(core_axis_name="c", subcore_axis_name="s")

    @functools.partial(
        pl.kernel,
        out_type=jax.ShapeDtypeStruct((B, D), dtype),
        mesh=mesh,
        scratch_types=[
            pltpu.VMEM((rows_per_w, _LANE), jnp.int32),
            pltpu.VMEM((_NBUF, chunk, D), dtype),
            pltpu.SemaphoreType.DMA,
            pltpu.SemaphoreType.DMA,
            pltpu.SemaphoreType.DMA,
            pltpu.SemaphoreType.DMA,
        ],
        compiler_params=pltpu.CompilerParams(use_tc_tiling_on_sc=False),
    )
    def gather_kernel(idx_hbm, table_hbm, out_hbm, idx_v, rows_v,
                      gsem0, gsem1, osem0, osem1):
        gsems = (gsem0, gsem1)
        osems = (osem0, osem1)
        wid = lax.axis_index("s") * _NC + lax.axis_index("c")
        out_base = wid * b_per_w

        pltpu.sync_copy(idx_hbm.at[pl.ds(wid * rows_per_w, rows_per_w)], idx_v)

        def fire(b, g):
            for j in range(_ROWS_PER_STEP):
                pltpu.async_copy(
                    table_hbm.at[idx_v.at[g * _ROWS_PER_STEP + j]],
                    rows_v.at[b, pl.ds(j * _LANE, _LANE)],
                    gsems[b],
                )

        def wait_gather(b):
            pltpu.make_async_copy(
                out_hbm.at[pl.ds(0, chunk)], rows_v.at[b], gsems[b]
            ).wait()

        def start_writeback(b, g):
            pltpu.async_copy(
                rows_v.at[b],
                out_hbm.at[pl.ds(out_base + g * chunk, chunk)],
                osems[b],
            )

        def wait_writeback(b):
            pltpu.make_async_copy(
                rows_v.at[b], out_hbm.at[pl.ds(0, chunk)], osems[b]
            ).wait()

        for b in range(_NBUF):
            fire(b, b)

        def outer(t, carry):
            g0 = t * _NBUF
            for b in range(_NBUF):
                wait_gather(b)
                start_writeback(b, g0 + b)
            for b in range(_NBUF):
                wait_writeback(b)
                fire(b, g0 + _NBUF + b)
            return carry

        lax.fori_loop(0, n_outer - 1, outer, 0)

        g0 = (n_outer - 1) * _NBUF
        for b in range(_NBUF):
            wait_gather(b)
            start_writeback(b, g0 + b)
        for b in range(_NBUF):
            wait_writeback(b)

    return gather_kernel


def kernel(walks, table):
    (B,) = walks.shape
    V, D = table.shape
    idx2d = walks.astype(jnp.int32).reshape(B // _LANE, _LANE)
    return _make_gather(B, V, D, table.dtype)(idx2d, table)
